# baseline (speedup 1.0000x reference)
import functools

import jax
import jax.numpy as jnp
from jax import lax
from jax.experimental import pallas as pl
from jax.experimental.pallas import tpu as pltpu
from jax.experimental.pallas import tpu_sc as plsc

_EPS = 1e-10
_NC = 2
_NS = 16
_CH = 64

_f32 = jnp.float32
_SC_PARAMS = pltpu.CompilerParams(use_tc_tiling_on_sc=False)
_MESH = plsc.VectorSubcoreMesh(core_axis_name="c", subcore_axis_name="s")


def _dense_total_tc(vals, t00, block):
    n = vals.shape[0]
    nb = n // block
    assert nb * block == n

    def body(v_ref, w_ref, d_ref, tot_ref):
        i = pl.program_id(0)
        blk = v_ref[...]
        d_ref[...] = jnp.dot(blk, w_ref[...], preferred_element_type=_f32)

        @pl.when(i == 0)
        def _():
            tot_ref[...] = jnp.zeros_like(tot_ref)

        tot_ref[...] += jnp.sum(blk, axis=0, keepdims=True)

    return pl.pallas_call(
        body,
        grid=(nb,),
        in_specs=[
            pl.BlockSpec((block, 32), lambda i: (i, 0)),
            pl.BlockSpec((32, 32), lambda i: (0, 0)),
        ],
        out_specs=[
            pl.BlockSpec((block, 32), lambda i: (i, 0)),
            pl.BlockSpec((1, 32), lambda i: (0, 0)),
        ],
        out_shape=[
            jax.ShapeDtypeStruct((n, 32), _f32),
            jax.ShapeDtypeStruct((1, 32), _f32),
        ],
    )(vals, t00)


def _segsum_sc(vals, row3, col3, z32, z8, ones8, num_seg):
    n = vals.shape[0]
    total_ch = n // _CH
    assert total_ch * _CH == n
    rows = num_seg // _NS
    assert rows * _NS == num_seg and rows % 8 == 0

    @functools.partial(
        pl.kernel,
        out_type=[
            jax.ShapeDtypeStruct((num_seg, 32), _f32),
            jax.ShapeDtypeStruct((num_seg, 8), _f32),
            jax.ShapeDtypeStruct((num_seg, 32), _f32),
            jax.ShapeDtypeStruct((num_seg, 8), _f32),
        ],
        mesh=_MESH,
        scratch_types=[
            pltpu.VMEM_SHARED((num_seg, 32), _f32),
            pltpu.VMEM_SHARED((num_seg, 8), _f32),
            pltpu.VMEM((_CH, 32), _f32),
            pltpu.VMEM((_CH, 32), _f32),
            pltpu.VMEM((1, _CH), jnp.int32),
            pltpu.VMEM((1, _CH), jnp.int32),
            pltpu.VMEM((_CH, 8), _f32),
            pltpu.SemaphoreType.DMA((2,)),
            pltpu.SemaphoreType.DMA((2,)),
        ],
        compiler_params=_SC_PARAMS,
    )
    def k(vals_hbm, row_hbm, col_hbm, z32_hbm, z8_hbm, ones_hbm,
          rs_hbm, rc_hbm, cs_hbm, cc_hbm, acc_s, acc_c,
          vbuf0, vbuf1, ibuf0, ibuf1, obuf, lsem, ssem):
        c = lax.axis_index("c")
        s = lax.axis_index("s")
        vbufs, ibufs = (vbuf0, vbuf1), (ibuf0, ibuf1)
        sl = pl.ds(s * rows, rows)
        pltpu.sync_copy(z32_hbm, acc_s.at[sl])
        pltpu.sync_copy(z8_hbm, acc_c.at[sl])
        pltpu.sync_copy(ones_hbm, obuf)
        plsc.subcore_barrier()

        def scan_axis(seg_hbm):
            nj = (total_ch - s + _NS - 1) // _NS

            def vals_cp(kk, b):
                base = (s + kk * _NS) * _CH
                return pltpu.make_async_copy(
                    vals_hbm.at[pl.ds(base, _CH)], vbufs[b], lsem.at[b])

            def seg_cp(kk, b):
                return pltpu.make_async_copy(
                    seg_hbm.at[s + kk * _NS], ibufs[b], lsem.at[b])

            def start_load(kk, b):
                vals_cp(kk, b).start()
                seg_cp(kk, b).start()

            @pl.when(nj > 0)
            def _():
                start_load(0, 0)

            @pl.when(nj > 1)
            def _():
                start_load(1, 1)

            @pl.loop(0, 2 * ((nj + 1) // 2), step=2)
            def _(j):
                for b in range(2):
                    kk = j + b

                    @pl.when(kk < nj)
                    def _():
                        vals_cp(kk, b).wait()
                        seg_cp(kk, b).wait()
                        idx = ibufs[b].at[0]
                        cp_s = pltpu.make_async_copy(
                            vbufs[b], acc_s.at[idx], ssem.at[b])
                        cp_c = pltpu.make_async_copy(
                            obuf, acc_c.at[idx], ssem.at[b])
                        cp_s.start(add=True)
                        cp_c.start(add=True)
                        cp_s.wait()
                        cp_c.wait()

                        @pl.when(kk + 2 < nj)
                        def _():
                            start_load(kk + 2, b)

        @pl.when(c == 0)
        def _():
            scan_axis(row_hbm)

        @pl.when(c == 1)
        def _():
            scan_axis(col_hbm)

        plsc.subcore_barrier()

        @pl.when(c == 0)
        def _():
            pltpu.sync_copy(acc_s.at[sl], rs_hbm.at[sl])
            pltpu.sync_copy(acc_c.at[sl], rc_hbm.at[sl])

        @pl.when(c == 1)
        def _():
            pltpu.sync_copy(acc_s.at[sl], cs_hbm.at[sl])
            pltpu.sync_copy(acc_c.at[sl], cc_hbm.at[sl])

    return k(vals, row3, col3, z32, z8, ones8)


def _marg_tc(rs, rc, cs, cc, total, t01, t10, t11, tb2, n_entries):
    num_seg = rs.shape[0]
    block = num_seg // 16
    assert block * 16 == num_seg and block % 8 == 0

    def body(rs_ref, rc_ref, cs_ref, cc_ref, tot_ref,
             t01_ref, t10_ref, t11_ref, tb_ref, g1_ref, g0_ref):
        m1 = rs_ref[...] / (rc_ref[:, 0:1] + _EPS)
        g1_ref[...] = jnp.dot(m1, t01_ref[...], preferred_element_type=_f32)
        m0 = cs_ref[...] / (cc_ref[:, 0:1] + _EPS)
        base = (jnp.dot(tot_ref[...] / n_entries, t11_ref[...],
                        preferred_element_type=_f32) + tb_ref[...])
        g0_ref[...] = (jnp.dot(m0, t10_ref[...], preferred_element_type=_f32)
                       + base)

    w_spec = pl.BlockSpec((32, 32), lambda i: (0, 0))
    v_spec = pl.BlockSpec((1, 32), lambda i: (0, 0))
    return pl.pallas_call(
        body,
        grid=(16,),
        in_specs=[
            pl.BlockSpec((block, 32), lambda i: (i, 0)),
            pl.BlockSpec((block, 8), lambda i: (i, 0)),
            pl.BlockSpec((block, 32), lambda i: (i, 0)),
            pl.BlockSpec((block, 8), lambda i: (i, 0)),
            v_spec, w_spec, w_spec, w_spec, v_spec,
        ],
        out_specs=[
            pl.BlockSpec((block, 32), lambda i: (i, 0)),
            pl.BlockSpec((block, 32), lambda i: (i, 0)),
        ],
        out_shape=[
            jax.ShapeDtypeStruct((num_seg, 32), _f32),
            jax.ShapeDtypeStruct((num_seg, 32), _f32),
        ],
    )(rs, rc, cs, cc, total, t01, t10, t11, tb2)


def _apply_sc(dense, row3, col3, g1, g0c):
    n = dense.shape[0]
    total_ch = n // _CH
    assert total_ch * _CH == n
    nw = _NC * _NS

    @functools.partial(
        pl.kernel,
        out_type=jax.ShapeDtypeStruct((n, 32), _f32),
        mesh=_MESH,
        scratch_types=[
            [pltpu.VMEM((_CH, 32), _f32)] * 2,
            [pltpu.VMEM((_CH, 32), _f32)] * 2,
            [pltpu.VMEM((_CH, 32), _f32)] * 2,
            [pltpu.VMEM((_CH, 32), _f32)] * 2,
            [pltpu.VMEM((1, _CH), jnp.int32)] * 2,
            [pltpu.VMEM((1, _CH), jnp.int32)] * 2,
            pltpu.SemaphoreType.DMA((2,)),
            pltpu.SemaphoreType.DMA((2,)),
            pltpu.SemaphoreType.DMA((2,)),
            pltpu.SemaphoreType.DMA((2,)),
        ],
        compiler_params=_SC_PARAMS,
    )
    def k(dense_hbm, row_hbm, col_hbm, g1_hbm, g0_hbm, out_hbm,
          dbufs, abufs, bbufs, ovs, rbufs, cbufs, dsem, isem, gsem, osem):
        c = lax.axis_index("c")
        s = lax.axis_index("s")
        w = s * _NC + c
        nj = (total_ch - w + nw - 1) // nw

        def dense_cp(kk, b):
            base = (w + kk * nw) * _CH
            return pltpu.make_async_copy(
                dense_hbm.at[pl.ds(base, _CH)], dbufs[b], dsem.at[b])

        def ridx_cp(kk, b):
            return pltpu.make_async_copy(
                row_hbm.at[w + kk * nw], rbufs[b], isem.at[b])

        def cidx_cp(kk, b):
            return pltpu.make_async_copy(
                col_hbm.at[w + kk * nw], cbufs[b], isem.at[b])

        def ga_cp(b):
            return pltpu.make_async_copy(
                g1_hbm.at[rbufs[b].at[0]], abufs[b], gsem.at[b])

        def gb_cp(b):
            return pltpu.make_async_copy(
                g0_hbm.at[cbufs[b].at[0]], bbufs[b], gsem.at[b])

        def out_cp(kk, b):
            base = (w + kk * nw) * _CH
            return pltpu.make_async_copy(
                ovs[b], out_hbm.at[pl.ds(base, _CH)], osem.at[b])

        def start_load(kk, b):
            dense_cp(kk, b).start()
            ridx_cp(kk, b).start()
            cidx_cp(kk, b).start()

        @pl.when(nj > 0)
        def _():
            start_load(0, 0)

        @pl.when(nj > 1)
        def _():
            start_load(1, 1)

        @pl.when(nj > 0)
        def _():
            ridx_cp(0, 0).wait()
            cidx_cp(0, 0).wait()
            ga_cp(0).start()
            gb_cp(0).start()

        @pl.loop(0, 2 * ((nj + 1) // 2), step=2)
        def _(j):
            for b in range(2):
                kk = j + b

                @pl.when(kk < nj)
                def _():
                    @pl.when(kk + 1 < nj)
                    def _():
                        ridx_cp(kk + 1, 1 - b).wait()
                        cidx_cp(kk + 1, 1 - b).wait()
                        ga_cp(1 - b).start()
                        gb_cp(1 - b).start()

                    dense_cp(kk, b).wait()
                    ga_cp(b).wait()
                    gb_cp(b).wait()

                    @pl.when(kk >= 2)
                    def _():
                        out_cp(kk - 2, b).wait()

                    @pl.loop(0, _CH)
                    def _(i):
                        for h in range(2):
                            hs = pl.ds(h * 16, 16)
                            ovs[b][i, hs] = jnp.maximum(
                                dbufs[b][i, hs] + abufs[b][i, hs]
                                + bbufs[b][i, hs], 0.0)

                    out_cp(kk, b).start()

                    @pl.when(kk + 2 < nj)
                    def _():
                        start_load(kk + 2, b)

        for b in range(2):
            m = ((nj - 1 - b) // 2) * 2 + b

            @pl.when(m >= 0)
            def _():
                out_cp(m, b).wait()

    return k(dense, row3, col3, g1, g0c)


def _split_idx_tc(inds, block):
    n = inds.shape[0]
    nb = n // block
    assert nb * block == n and block % _CH == 0

    def body(i_ref, r_ref, c_ref):
        x = i_ref[...]
        r_ref[...] = x[:, 0].reshape(block // _CH, 1, _CH)
        c_ref[...] = x[:, 1].reshape(block // _CH, 1, _CH)

    return pl.pallas_call(
        body,
        grid=(nb,),
        in_specs=[pl.BlockSpec((block, 2), lambda i: (i, 0))],
        out_specs=[
            pl.BlockSpec((block // _CH, 1, _CH), lambda i: (i, 0, 0)),
            pl.BlockSpec((block // _CH, 1, _CH), lambda i: (i, 0, 0)),
        ],
        out_shape=[
            jax.ShapeDtypeStruct((n // _CH, 1, _CH), jnp.int32),
            jax.ShapeDtypeStruct((n // _CH, 1, _CH), jnp.int32),
        ],
    )(inds)


def _table(vals, inds, num_seg, t00, t01, t10, t11, tb2, blk):
    n = vals.shape[0]
    row3, col3 = _split_idx_tc(inds, 8000)
    z32 = jnp.zeros((num_seg // _NS, 32), _f32)
    z8 = jnp.zeros((num_seg // _NS, 8), _f32)
    ones8 = jnp.ones((_CH, 8), _f32)
    dense, total = _dense_total_tc(vals, t00, blk)
    rs, rc, cs, cc = _segsum_sc(vals, row3, col3, z32, z8, ones8, num_seg)
    g1, g0c = _marg_tc(rs, rc, cs, cc, total, t01, t10, t11, tb2, float(n))
    return _apply_sc(dense, row3, col3, g1, g0c)


def kernel(t0_values, t0_indices, t1_values, t1_indices, t2_values,
           t2_indices, t0_theta_00, t0_theta_01, t0_theta_10, t0_theta_11,
           t1_theta_00, t1_theta_01, t1_theta_10, t1_theta_11,
           t2_theta_00, t2_theta_01, t2_theta_10, t2_theta_11, theta_b):
    tb2 = theta_b.reshape(1, 32)
    o0 = _table(t0_values, t0_indices, 50176,
                t0_theta_00, t0_theta_01, t0_theta_10, t0_theta_11, tb2, 8000)
    o1 = _table(t1_values, t1_indices, 10112,
                t1_theta_00, t1_theta_01, t1_theta_10, t1_theta_11, tb2, 8000)
    o2 = _table(t2_values, t2_indices, 10112,
                t2_theta_00, t2_theta_01, t2_theta_10, t2_theta_11, tb2, 8000)
    return (o0, o1, o2)


# speedup vs baseline: 1.2729x; 1.2729x over previous
import functools

import jax
import jax.numpy as jnp
from jax import lax
from jax.experimental import pallas as pl
from jax.experimental.pallas import tpu as pltpu
from jax.experimental.pallas import tpu_sc as plsc

_EPS = 1e-10
_NC = 2
_NS = 16
_CH = 64

_f32 = jnp.float32
_SC_PARAMS = pltpu.CompilerParams(use_tc_tiling_on_sc=False)
_MESH = plsc.VectorSubcoreMesh(core_axis_name="c", subcore_axis_name="s")


def _dense_total_tc(vals, t00, block):
    n = vals.shape[0]
    nb = n // block
    assert nb * block == n

    def body(v_ref, w_ref, d_ref, tot_ref):
        i = pl.program_id(0)
        blk = v_ref[...]
        d_ref[...] = jnp.dot(blk, w_ref[...], preferred_element_type=_f32)

        @pl.when(i == 0)
        def _():
            tot_ref[...] = jnp.zeros_like(tot_ref)

        tot_ref[...] += jnp.sum(blk, axis=0, keepdims=True)

    return pl.pallas_call(
        body,
        grid=(nb,),
        in_specs=[
            pl.BlockSpec((block, 32), lambda i: (i, 0)),
            pl.BlockSpec((32, 32), lambda i: (0, 0)),
        ],
        out_specs=[
            pl.BlockSpec((block, 32), lambda i: (i, 0)),
            pl.BlockSpec((1, 32), lambda i: (0, 0)),
        ],
        out_shape=[
            jax.ShapeDtypeStruct((n, 32), _f32),
            jax.ShapeDtypeStruct((1, 32), _f32),
        ],
    )(vals, t00)


def _segsum_sc(vals, row3, col3, z32, z8, ones8, num_seg):
    n = vals.shape[0]
    total_ch = n // _CH
    assert total_ch * _CH == n
    rows = num_seg // _NS
    assert rows * _NS == num_seg and rows % 8 == 0

    @functools.partial(
        pl.kernel,
        out_type=[
            jax.ShapeDtypeStruct((num_seg, 32), _f32),
            jax.ShapeDtypeStruct((num_seg, 8), _f32),
            jax.ShapeDtypeStruct((num_seg, 32), _f32),
            jax.ShapeDtypeStruct((num_seg, 8), _f32),
        ],
        mesh=_MESH,
        scratch_types=[
            pltpu.VMEM_SHARED((num_seg, 32), _f32),
            pltpu.VMEM_SHARED((num_seg, 8), _f32),
            pltpu.VMEM((_CH, 32), _f32),
            pltpu.VMEM((_CH, 32), _f32),
            pltpu.VMEM((1, _CH), jnp.int32),
            pltpu.VMEM((1, _CH), jnp.int32),
            pltpu.VMEM((_CH, 8), _f32),
            pltpu.SemaphoreType.DMA((2,)),
            pltpu.SemaphoreType.DMA((2,)),
        ],
        compiler_params=_SC_PARAMS,
    )
    def k(vals_hbm, row_hbm, col_hbm, z32_hbm, z8_hbm, ones_hbm,
          rs_hbm, rc_hbm, cs_hbm, cc_hbm, acc_s, acc_c,
          vbuf0, vbuf1, ibuf0, ibuf1, obuf, lsem, ssem):
        c = lax.axis_index("c")
        s = lax.axis_index("s")
        vbufs, ibufs = (vbuf0, vbuf1), (ibuf0, ibuf1)
        sl = pl.ds(s * rows, rows)
        pltpu.sync_copy(z32_hbm, acc_s.at[sl])
        pltpu.sync_copy(z8_hbm, acc_c.at[sl])
        pltpu.sync_copy(ones_hbm, obuf)
        plsc.subcore_barrier()

        def scan_axis(seg_hbm):
            nj = (total_ch - s + _NS - 1) // _NS

            def vals_cp(kk, b):
                base = (s + kk * _NS) * _CH
                return pltpu.make_async_copy(
                    vals_hbm.at[pl.ds(base, _CH)], vbufs[b], lsem.at[b])

            def seg_cp(kk, b):
                return pltpu.make_async_copy(
                    seg_hbm.at[s + kk * _NS], ibufs[b], lsem.at[b])

            def start_load(kk, b):
                vals_cp(kk, b).start()
                seg_cp(kk, b).start()

            @pl.when(nj > 0)
            def _():
                start_load(0, 0)

            @pl.when(nj > 1)
            def _():
                start_load(1, 1)

            @pl.loop(0, 2 * ((nj + 1) // 2), step=2)
            def _(j):
                for b in range(2):
                    kk = j + b

                    @pl.when(kk < nj)
                    def _():
                        vals_cp(kk, b).wait()
                        seg_cp(kk, b).wait()
                        idx = ibufs[b].at[0]
                        cp_s = pltpu.make_async_copy(
                            vbufs[b], acc_s.at[idx], ssem.at[b])
                        cp_c = pltpu.make_async_copy(
                            obuf, acc_c.at[idx], ssem.at[b])
                        cp_s.start(add=True)
                        cp_c.start(add=True)
                        cp_s.wait()
                        cp_c.wait()

                        @pl.when(kk + 2 < nj)
                        def _():
                            start_load(kk + 2, b)

        @pl.when(c == 0)
        def _():
            scan_axis(row_hbm)

        @pl.when(c == 1)
        def _():
            scan_axis(col_hbm)

        plsc.subcore_barrier()

        @pl.when(c == 0)
        def _():
            pltpu.sync_copy(acc_s.at[sl], rs_hbm.at[sl])
            pltpu.sync_copy(acc_c.at[sl], rc_hbm.at[sl])

        @pl.when(c == 1)
        def _():
            pltpu.sync_copy(acc_s.at[sl], cs_hbm.at[sl])
            pltpu.sync_copy(acc_c.at[sl], cc_hbm.at[sl])

    return k(vals, row3, col3, z32, z8, ones8)


def _marg_tc(rs, rc, cs, cc, total, t01, t10, t11, tb2, n_entries):
    num_seg = rs.shape[0]
    block = num_seg // 16
    assert block * 16 == num_seg and block % 8 == 0

    def body(rs_ref, rc_ref, cs_ref, cc_ref, tot_ref,
             t01_ref, t10_ref, t11_ref, tb_ref, g1_ref, g0_ref):
        m1 = rs_ref[...] / (rc_ref[:, 0:1] + _EPS)
        g1_ref[...] = jnp.dot(m1, t01_ref[...], preferred_element_type=_f32)
        m0 = cs_ref[...] / (cc_ref[:, 0:1] + _EPS)
        base = (jnp.dot(tot_ref[...] / n_entries, t11_ref[...],
                        preferred_element_type=_f32) + tb_ref[...])
        g0_ref[...] = (jnp.dot(m0, t10_ref[...], preferred_element_type=_f32)
                       + base)

    w_spec = pl.BlockSpec((32, 32), lambda i: (0, 0))
    v_spec = pl.BlockSpec((1, 32), lambda i: (0, 0))
    return pl.pallas_call(
        body,
        grid=(16,),
        in_specs=[
            pl.BlockSpec((block, 32), lambda i: (i, 0)),
            pl.BlockSpec((block, 8), lambda i: (i, 0)),
            pl.BlockSpec((block, 32), lambda i: (i, 0)),
            pl.BlockSpec((block, 8), lambda i: (i, 0)),
            v_spec, w_spec, w_spec, w_spec, v_spec,
        ],
        out_specs=[
            pl.BlockSpec((block, 32), lambda i: (i, 0)),
            pl.BlockSpec((block, 32), lambda i: (i, 0)),
        ],
        out_shape=[
            jax.ShapeDtypeStruct((num_seg, 32), _f32),
            jax.ShapeDtypeStruct((num_seg, 32), _f32),
        ],
    )(rs, rc, cs, cc, total, t01, t10, t11, tb2)


def _apply_sc(dense, row3, col3, g1, g0c):
    n = dense.shape[0]
    total_ch = n // _CH
    assert total_ch * _CH == n
    nw = _NC * _NS

    @functools.partial(
        pl.kernel,
        out_type=jax.ShapeDtypeStruct((n, 32), _f32),
        mesh=_MESH,
        scratch_types=[
            [pltpu.VMEM((_CH, 32), _f32)] * 2,
            [pltpu.VMEM((_CH, 32), _f32)] * 2,
            [pltpu.VMEM((_CH, 32), _f32)] * 2,
            [pltpu.VMEM((_CH, 32), _f32)] * 2,
            [pltpu.VMEM((1, _CH), jnp.int32)] * 2,
            [pltpu.VMEM((1, _CH), jnp.int32)] * 2,
            pltpu.SemaphoreType.DMA((2,)),
            pltpu.SemaphoreType.DMA((2,)),
            pltpu.SemaphoreType.DMA((2,)),
            pltpu.SemaphoreType.DMA((2,)),
        ],
        compiler_params=_SC_PARAMS,
    )
    def k(dense_hbm, row_hbm, col_hbm, g1_hbm, g0_hbm, out_hbm,
          dbufs, abufs, bbufs, ovs, rbufs, cbufs, dsem, isem, gsem, osem):
        c = lax.axis_index("c")
        s = lax.axis_index("s")
        w = s * _NC + c
        nj = (total_ch - w + nw - 1) // nw

        def dense_cp(kk, b):
            base = (w + kk * nw) * _CH
            return pltpu.make_async_copy(
                dense_hbm.at[pl.ds(base, _CH)], dbufs[b], dsem.at[b])

        def ridx_cp(kk, b):
            return pltpu.make_async_copy(
                row_hbm.at[w + kk * nw], rbufs[b], isem.at[b])

        def cidx_cp(kk, b):
            return pltpu.make_async_copy(
                col_hbm.at[w + kk * nw], cbufs[b], isem.at[b])

        def ga_cp(b):
            return pltpu.make_async_copy(
                g1_hbm.at[rbufs[b].at[0]], abufs[b], gsem.at[b])

        def gb_cp(b):
            return pltpu.make_async_copy(
                g0_hbm.at[cbufs[b].at[0]], bbufs[b], gsem.at[b])

        def out_cp(kk, b):
            base = (w + kk * nw) * _CH
            return pltpu.make_async_copy(
                ovs[b], out_hbm.at[pl.ds(base, _CH)], osem.at[b])

        def start_load(kk, b):
            dense_cp(kk, b).start()
            ridx_cp(kk, b).start()
            cidx_cp(kk, b).start()

        @pl.when(nj > 0)
        def _():
            start_load(0, 0)

        @pl.when(nj > 1)
        def _():
            start_load(1, 1)

        @pl.when(nj > 0)
        def _():
            ridx_cp(0, 0).wait()
            cidx_cp(0, 0).wait()
            ga_cp(0).start()
            gb_cp(0).start()

        @pl.loop(0, 2 * ((nj + 1) // 2), step=2)
        def _(j):
            for b in range(2):
                kk = j + b

                @pl.when(kk < nj)
                def _():
                    @pl.when(kk + 1 < nj)
                    def _():
                        ridx_cp(kk + 1, 1 - b).wait()
                        cidx_cp(kk + 1, 1 - b).wait()
                        ga_cp(1 - b).start()
                        gb_cp(1 - b).start()

                    dense_cp(kk, b).wait()
                    ga_cp(b).wait()
                    gb_cp(b).wait()

                    @pl.when(kk >= 2)
                    def _():
                        out_cp(kk - 2, b).wait()

                    @pl.loop(0, _CH)
                    def _(i):
                        for h in range(2):
                            hs = pl.ds(h * 16, 16)
                            ovs[b][i, hs] = jnp.maximum(
                                dbufs[b][i, hs] + abufs[b][i, hs]
                                + bbufs[b][i, hs], 0.0)

                    out_cp(kk, b).start()

                    @pl.when(kk + 2 < nj)
                    def _():
                        start_load(kk + 2, b)

        for b in range(2):
            m = ((nj - 1 - b) // 2) * 2 + b

            @pl.when(m >= 0)
            def _():
                out_cp(m, b).wait()

    return k(dense, row3, col3, g1, g0c)


def _table(vals, inds, num_seg, t00, t01, t10, t11, tb2, blk):
    n = vals.shape[0]
    row3 = inds[:, 0].reshape(n // _CH, 1, _CH)
    col3 = inds[:, 1].reshape(n // _CH, 1, _CH)
    z32 = jnp.zeros((num_seg // _NS, 32), _f32)
    z8 = jnp.zeros((num_seg // _NS, 8), _f32)
    ones8 = jnp.ones((_CH, 8), _f32)
    dense, total = _dense_total_tc(vals, t00, blk)
    rs, rc, cs, cc = _segsum_sc(vals, row3, col3, z32, z8, ones8, num_seg)
    g1, g0c = _marg_tc(rs, rc, cs, cc, total, t01, t10, t11, tb2, float(n))
    return _apply_sc(dense, row3, col3, g1, g0c)


def kernel(t0_values, t0_indices, t1_values, t1_indices, t2_values,
           t2_indices, t0_theta_00, t0_theta_01, t0_theta_10, t0_theta_11,
           t1_theta_00, t1_theta_01, t1_theta_10, t1_theta_11,
           t2_theta_00, t2_theta_01, t2_theta_10, t2_theta_11, theta_b):
    tb2 = theta_b.reshape(1, 32)
    o0 = _table(t0_values, t0_indices, 50176,
                t0_theta_00, t0_theta_01, t0_theta_10, t0_theta_11, tb2, 8000)
    o1 = _table(t1_values, t1_indices, 10112,
                t1_theta_00, t1_theta_01, t1_theta_10, t1_theta_11, tb2, 8000)
    o2 = _table(t2_values, t2_indices, 10112,
                t2_theta_00, t2_theta_01, t2_theta_10, t2_theta_11, tb2, 8000)
    return (o0, o1, o2)
